# baseline (speedup 1.0000x reference)
import functools

import jax
import jax.numpy as jnp
from jax import lax
from jax.experimental import pallas as pl
from jax.experimental.pallas import tpu as pltpu
from jax.experimental.pallas import tpu_sc as plsc

E = 8
TOPK = 2
T = 2048
H = 2048
I = 1408
BR = 256
NP = T * TOPK + E * BR
NB = NP // BR
NBPAD = 64


def _shift_down(a, k):
    return jnp.concatenate([jnp.zeros((k, a.shape[1]), a.dtype), a[:-k]], axis=0)


def _meta_body(logits_ref, pos0_ref, pos1_ref, w0b_ref, w1b_ref, be_ref):
    logits = logits_ref[...]
    m = jnp.max(logits, axis=-1, keepdims=True)
    ex = jnp.exp(logits - m)
    gates = ex / jnp.sum(ex, axis=-1, keepdims=True)
    cols = jax.lax.broadcasted_iota(jnp.int32, gates.shape, 1)
    m1 = jnp.max(gates, axis=-1, keepdims=True)
    a1 = jnp.min(jnp.where(gates >= m1, cols, E), axis=-1, keepdims=True)
    mask1 = cols == a1
    g2 = jnp.where(mask1, -jnp.inf, gates)
    m2 = jnp.max(g2, axis=-1, keepdims=True)
    a2 = jnp.min(jnp.where(g2 >= m2, cols, E), axis=-1, keepdims=True)
    mask2 = cols == a2
    denom = m1 + m2
    w0 = m1 / denom
    w1r = m2 / denom

    oh = mask1.astype(jnp.int32) + mask2.astype(jnp.int32)
    incl = oh
    k = 1
    while k < T:
        incl = incl + _shift_down(incl, k)
        k *= 2
    excl = incl - oh
    counts = incl[T - 1:T, :]
    padded = ((counts + BR - 1) // BR) * BR
    incl8 = padded
    k = 1
    while k < E:
        incl8 = incl8 + jnp.concatenate(
            [jnp.zeros((1, k), jnp.int32), incl8[:, :-k]], axis=1)
        k *= 2
    po = incl8 - padded

    rank0 = jnp.sum(jnp.where(mask1, excl, 0), axis=-1, keepdims=True)
    rank1 = jnp.sum(jnp.where(mask2, excl, 0), axis=-1, keepdims=True)
    off0 = jnp.sum(jnp.where(mask1, po, 0), axis=-1, keepdims=True)
    off1 = jnp.sum(jnp.where(mask2, po, 0), axis=-1, keepdims=True)
    pos0_ref[...] = off0 + rank0
    pos1_ref[...] = off1 + rank1
    w0b_ref[...] = jnp.broadcast_to(w0, (T, 128))
    w1b_ref[...] = jnp.broadcast_to(w1r, (T, 128))

    bidx = jax.lax.broadcasted_iota(jnp.int32, (NBPAD, E), 0) * BR
    po_b = jnp.broadcast_to(po, (NBPAD, E))
    be = jnp.sum((bidx >= po_b).astype(jnp.int32), axis=-1, keepdims=True) - 1
    be_ref[...] = jnp.clip(be, 0, E - 1)


def _meta_call(router_logits):
    return pl.pallas_call(
        _meta_body,
        out_shape=[
            jax.ShapeDtypeStruct((T, 1), jnp.int32),
            jax.ShapeDtypeStruct((T, 1), jnp.int32),
            jax.ShapeDtypeStruct((T, 128), jnp.float32),
            jax.ShapeDtypeStruct((T, 128), jnp.float32),
            jax.ShapeDtypeStruct((NBPAD, 1), jnp.int32),
        ],
    )(router_logits)


def _mlp_body(be_ref, xs_ref, w1_ref, w2_ref, ys_ref):
    h = jax.lax.dot_general(xs_ref[...], w1_ref[0], (((1,), (1,)), ((), ())),
                            preferred_element_type=jnp.float32,
                            precision=jax.lax.Precision.DEFAULT)
    h = h * jax.nn.sigmoid(h)
    y = jax.lax.dot_general(h, w2_ref[0], (((1,), (1,)), ((), ())),
                            preferred_element_type=jnp.float32,
                            precision=jax.lax.Precision.DEFAULT)
    ys_ref[...] = y


def _mlp_call(be, xs, w1, w2):
    grid_spec = pltpu.PrefetchScalarGridSpec(
        num_scalar_prefetch=1,
        grid=(NB,),
        in_specs=[
            pl.BlockSpec((BR, H), lambda b, be_s: (b, 0)),
            pl.BlockSpec((1, I, H), lambda b, be_s: (be_s[b, 0], 0, 0)),
            pl.BlockSpec((1, H, I), lambda b, be_s: (be_s[b, 0], 0, 0)),
        ],
        out_specs=pl.BlockSpec((BR, H), lambda b, be_s: (b, 0)),
    )
    return pl.pallas_call(
        _mlp_body,
        grid_spec=grid_spec,
        out_shape=jax.ShapeDtypeStruct((NP, H), jnp.float32),
    )(be, xs, w1, w2)


_SC_MESH = plsc.VectorSubcoreMesh(core_axis_name="c", subcore_axis_name="s")
_NC = 2
_NS = 16
_NW = _NC * _NS
_TPW = T // _NW
_DCH = 16
_CCH = 8


@functools.partial(
    pl.kernel,
    mesh=_SC_MESH,
    out_type=jax.ShapeDtypeStruct((NP, H), jnp.float32),
    scratch_types=[
        pltpu.VMEM((2, 2, _DCH), jnp.int32),
        pltpu.VMEM((2, _DCH, H), jnp.float32),
        pltpu.SemaphoreType.DMA,
        pltpu.SemaphoreType.DMA,
    ],
)
def _dispatch(x_hbm, p0_hbm, p1_hbm, xs_hbm,
              idx_v, rows_v, sem0, sem1):
    wid = lax.axis_index("s") * _NC + lax.axis_index("c")
    base = wid * _TPW
    nsub = _TPW // _DCH
    sems = (sem0, sem1)
    pend = [None, None]
    for sub in range(nsub):
        slot = sub % 2
        b = base + sub * _DCH
        if pend[slot] is not None:
            for cp in pend[slot]:
                cp.wait()
        idx = idx_v.at[slot]
        rows = rows_v.at[slot]
        pltpu.sync_copy(p0_hbm.at[pl.ds(b, _DCH)], idx.at[0])
        pltpu.sync_copy(p1_hbm.at[pl.ds(b, _DCH)], idx.at[1])
        pltpu.sync_copy(x_hbm.at[pl.ds(b, _DCH)], rows)
        pend[slot] = (
            pltpu.async_copy(rows, xs_hbm.at[idx.at[0]], sems[slot]),
            pltpu.async_copy(rows, xs_hbm.at[idx.at[1]], sems[slot]),
        )
    for p in pend:
        if p is not None:
            for cp in p:
                cp.wait()


_NSUB = _TPW // _CCH


@functools.partial(
    pl.kernel,
    mesh=_SC_MESH,
    out_type=jax.ShapeDtypeStruct((T, H), jnp.float32),
    scratch_types=[
        pltpu.VMEM((2, _TPW), jnp.int32),
        pltpu.VMEM((_TPW, 128), jnp.float32),
        pltpu.VMEM((_TPW, 128), jnp.float32),
        pltpu.VMEM((_CCH, H), jnp.float32),
        pltpu.VMEM((_CCH, H), jnp.float32),
        pltpu.VMEM((_CCH, H), jnp.float32),
        pltpu.VMEM((_CCH, H), jnp.float32),
        pltpu.VMEM((_CCH, H), jnp.float32),
        pltpu.SemaphoreType.DMA,
        pltpu.SemaphoreType.DMA,
    ],
)
def _combine(ys_hbm, p0_hbm, p1_hbm, w0_hbm, w1_hbm, out_hbm,
             idx_v, wv0, wv1, a0, b0, a1, b1, out_v, sem0, sem1):
    wid = lax.axis_index("s") * _NC + lax.axis_index("c")
    base = wid * _TPW
    pltpu.sync_copy(p0_hbm.at[pl.ds(base, _TPW)], idx_v.at[0])
    pltpu.sync_copy(p1_hbm.at[pl.ds(base, _TPW)], idx_v.at[1])
    pltpu.sync_copy(w0_hbm.at[pl.ds(base, _TPW)], wv0)
    pltpu.sync_copy(w1_hbm.at[pl.ds(base, _TPW)], wv1)
    bufs = ((a0, b0, sem0), (a1, b1, sem1))

    def _issue(g, slot):
        av, bv, sem = bufs[slot]
        sl = pl.ds(g * _CCH, _CCH)
        ca = pltpu.async_copy(ys_hbm.at[idx_v.at[0, sl]], av, sem)
        cb = pltpu.async_copy(ys_hbm.at[idx_v.at[1, sl]], bv, sem)
        return ca, cb

    pend = [None, None]
    pend[0] = _issue(0, 0)
    for g in range(_NSUB):
        slot = g % 2
        if g + 1 < _NSUB:
            pend[(g + 1) % 2] = _issue(g + 1, (g + 1) % 2)
        ca, cb = pend[slot]
        ca.wait()
        cb.wait()
        av, bv, _ = bufs[slot]
        for i in range(_CCH):
            wa = wv0[g * _CCH + i, 0:16]
            wb = wv1[g * _CCH + i, 0:16]

            @plsc.parallel_loop(0, H // 16, 1, unroll=16)
            def _col(j, i=i, wa=wa, wb=wb):
                sl = pl.ds(j * 16, 16)
                out_v[i, sl] = wa * av[i, sl] + wb * bv[i, sl]
        pltpu.sync_copy(out_v, out_hbm.at[pl.ds(base + g * _CCH, _CCH)])


@jax.jit
def kernel(x, router_logits, w1, w2):
    pos0c, pos1c, w0b, w1b, be = _meta_call(router_logits)
    pos0 = pos0c.reshape(T)
    pos1 = pos1c.reshape(T)
    xs = _dispatch(x, pos0, pos1)
    ys = _mlp_call(be, xs, w1, w2)
    return _combine(ys, pos0, pos1, w0b, w1b)


# speedup vs baseline: 1.0148x; 1.0148x over previous
import functools

import jax
import jax.numpy as jnp
from jax import lax
from jax.experimental import pallas as pl
from jax.experimental.pallas import tpu as pltpu
from jax.experimental.pallas import tpu_sc as plsc

E = 8
TOPK = 2
T = 2048
H = 2048
I = 1408
BR = 256
NP = T * TOPK + E * BR
NB = NP // BR
NBPAD = 64


def _shift_down(a, k):
    return jnp.concatenate([jnp.zeros((k, a.shape[1]), a.dtype), a[:-k]], axis=0)


def _meta_body(logits_ref, pos0_ref, pos1_ref, w0b_ref, w1b_ref, be_ref):
    logits = logits_ref[...]
    m = jnp.max(logits, axis=-1, keepdims=True)
    ex = jnp.exp(logits - m)
    gates = ex / jnp.sum(ex, axis=-1, keepdims=True)
    cols = jax.lax.broadcasted_iota(jnp.int32, gates.shape, 1)
    m1 = jnp.max(gates, axis=-1, keepdims=True)
    a1 = jnp.min(jnp.where(gates >= m1, cols, E), axis=-1, keepdims=True)
    mask1 = cols == a1
    g2 = jnp.where(mask1, -jnp.inf, gates)
    m2 = jnp.max(g2, axis=-1, keepdims=True)
    a2 = jnp.min(jnp.where(g2 >= m2, cols, E), axis=-1, keepdims=True)
    mask2 = cols == a2
    denom = m1 + m2
    w0 = m1 / denom
    w1r = m2 / denom

    oh = mask1.astype(jnp.int32) + mask2.astype(jnp.int32)
    incl = oh
    k = 1
    while k < T:
        incl = incl + _shift_down(incl, k)
        k *= 2
    excl = incl - oh
    counts = incl[T - 1:T, :]
    padded = ((counts + BR - 1) // BR) * BR
    incl8 = padded
    k = 1
    while k < E:
        incl8 = incl8 + jnp.concatenate(
            [jnp.zeros((1, k), jnp.int32), incl8[:, :-k]], axis=1)
        k *= 2
    po = incl8 - padded

    rank0 = jnp.sum(jnp.where(mask1, excl, 0), axis=-1, keepdims=True)
    rank1 = jnp.sum(jnp.where(mask2, excl, 0), axis=-1, keepdims=True)
    off0 = jnp.sum(jnp.where(mask1, po, 0), axis=-1, keepdims=True)
    off1 = jnp.sum(jnp.where(mask2, po, 0), axis=-1, keepdims=True)
    pos0_ref[...] = off0 + rank0
    pos1_ref[...] = off1 + rank1
    w0b_ref[...] = jnp.broadcast_to(w0, (T, 128))
    w1b_ref[...] = jnp.broadcast_to(w1r, (T, 128))

    bidx = jax.lax.broadcasted_iota(jnp.int32, (NBPAD, E), 0) * BR
    po_b = jnp.broadcast_to(po, (NBPAD, E))
    be = jnp.sum((bidx >= po_b).astype(jnp.int32), axis=-1, keepdims=True) - 1
    be_ref[...] = jnp.clip(be, 0, E - 1)


def _meta_call(router_logits):
    return pl.pallas_call(
        _meta_body,
        out_shape=[
            jax.ShapeDtypeStruct((T, 1), jnp.int32),
            jax.ShapeDtypeStruct((T, 1), jnp.int32),
            jax.ShapeDtypeStruct((T, 128), jnp.float32),
            jax.ShapeDtypeStruct((T, 128), jnp.float32),
            jax.ShapeDtypeStruct((NBPAD, 1), jnp.int32),
        ],
    )(router_logits)


def _mlp_body(be_ref, xs_ref, w1_ref, w2_ref, ys_ref):
    h = jax.lax.dot_general(xs_ref[...], w1_ref[0], (((1,), (1,)), ((), ())),
                            preferred_element_type=jnp.float32,
                            precision=jax.lax.Precision.DEFAULT)
    h = h * jax.nn.sigmoid(h)
    y = jax.lax.dot_general(h, w2_ref[0], (((1,), (1,)), ((), ())),
                            preferred_element_type=jnp.float32,
                            precision=jax.lax.Precision.DEFAULT)
    ys_ref[...] = y


def _mlp_call(be, xs, w1, w2):
    grid_spec = pltpu.PrefetchScalarGridSpec(
        num_scalar_prefetch=1,
        grid=(NB,),
        in_specs=[
            pl.BlockSpec((BR, H), lambda b, be_s: (b, 0)),
            pl.BlockSpec((1, I, H), lambda b, be_s: (be_s[b, 0], 0, 0)),
            pl.BlockSpec((1, H, I), lambda b, be_s: (be_s[b, 0], 0, 0)),
        ],
        out_specs=pl.BlockSpec((BR, H), lambda b, be_s: (b, 0)),
    )
    return pl.pallas_call(
        _mlp_body,
        grid_spec=grid_spec,
        out_shape=jax.ShapeDtypeStruct((NP, H), jnp.float32),
    )(be, xs, w1, w2)


_SC_MESH = plsc.VectorSubcoreMesh(core_axis_name="c", subcore_axis_name="s")
_NC = 2
_NS = 16
_NW = _NC * _NS
_TPW = T // _NW
_DCH = 16
_CCH = 8


@functools.partial(
    pl.kernel,
    mesh=_SC_MESH,
    out_type=jax.ShapeDtypeStruct((NP, H), jnp.float32),
    scratch_types=[
        pltpu.VMEM((2, 2, _DCH), jnp.int32),
        pltpu.VMEM((2, _DCH, H), jnp.float32),
        pltpu.SemaphoreType.DMA,
        pltpu.SemaphoreType.DMA,
    ],
)
def _dispatch(x_hbm, p0_hbm, p1_hbm, xs_hbm,
              idx_v, rows_v, sem0, sem1):
    wid = lax.axis_index("s") * _NC + lax.axis_index("c")
    base = wid * _TPW
    nsub = _TPW // _DCH
    sems = (sem0, sem1)
    pend = [None, None]
    for sub in range(nsub):
        slot = sub % 2
        b = base + sub * _DCH
        if pend[slot] is not None:
            for cp in pend[slot]:
                cp.wait()
        idx = idx_v.at[slot]
        rows = rows_v.at[slot]
        pltpu.sync_copy(p0_hbm.at[pl.ds(b, _DCH)], idx.at[0])
        pltpu.sync_copy(p1_hbm.at[pl.ds(b, _DCH)], idx.at[1])
        pltpu.sync_copy(x_hbm.at[pl.ds(b, _DCH)], rows)
        pend[slot] = (
            pltpu.async_copy(rows, xs_hbm.at[idx.at[0]], sems[slot]),
            pltpu.async_copy(rows, xs_hbm.at[idx.at[1]], sems[slot]),
        )
    for p in pend:
        if p is not None:
            for cp in p:
                cp.wait()


_NSUB = _TPW // _CCH


@functools.partial(
    pl.kernel,
    mesh=_SC_MESH,
    out_type=jax.ShapeDtypeStruct((T, H), jnp.float32),
    scratch_types=[
        pltpu.VMEM((2, _TPW), jnp.int32),
        pltpu.VMEM((_TPW, 128), jnp.float32),
        pltpu.VMEM((_TPW, 128), jnp.float32),
        pltpu.VMEM((_CCH, H), jnp.float32),
        pltpu.VMEM((_CCH, H), jnp.float32),
        pltpu.VMEM((_CCH, H), jnp.float32),
        pltpu.VMEM((_CCH, H), jnp.float32),
        pltpu.VMEM((_CCH, H), jnp.float32),
        pltpu.SemaphoreType.DMA,
        pltpu.SemaphoreType.DMA,
    ],
)
def _combine(ys_hbm, p0_hbm, p1_hbm, w0_hbm, w1_hbm, out_hbm,
             idx_v, wv0, wv1, a0, b0, a1, b1, out_v, sem0, sem1):
    wid = lax.axis_index("s") * _NC + lax.axis_index("c")
    base = wid * _TPW
    pltpu.sync_copy(p0_hbm.at[pl.ds(base, _TPW)], idx_v.at[0])
    pltpu.sync_copy(p1_hbm.at[pl.ds(base, _TPW)], idx_v.at[1])
    pltpu.sync_copy(w0_hbm.at[pl.ds(base, _TPW)], wv0)
    pltpu.sync_copy(w1_hbm.at[pl.ds(base, _TPW)], wv1)
    bufs = ((a0, b0, sem0), (a1, b1, sem1))

    def _issue(g, slot):
        av, bv, sem = bufs[slot]
        sl = pl.ds(g * _CCH, _CCH)
        ca = pltpu.async_copy(ys_hbm.at[idx_v.at[0, sl]], av, sem)
        cb = pltpu.async_copy(ys_hbm.at[idx_v.at[1, sl]], bv, sem)
        return ca, cb

    pend = [None, None]
    pend[0] = _issue(0, 0)
    for g in range(_NSUB):
        slot = g % 2
        if g + 1 < _NSUB:
            pend[(g + 1) % 2] = _issue(g + 1, (g + 1) % 2)
        ca, cb = pend[slot]
        ca.wait()
        cb.wait()
        av, bv, _ = bufs[slot]
        for i in range(_CCH):
            wa = wv0[g * _CCH + i, 0:16]
            wb = wv1[g * _CCH + i, 0:16]

            @plsc.parallel_loop(0, H // 16, 1, unroll=8)
            def _col(j, i=i, wa=wa, wb=wb):
                sl = pl.ds(j * 16, 16)
                out_v[i, sl] = wa * av[i, sl] + wb * bv[i, sl]
        pltpu.sync_copy(out_v, out_hbm.at[pl.ds(base + g * _CCH, _CCH)])


@jax.jit
def kernel(x, router_logits, w1, w2):
    pos0c, pos1c, w0b, w1b, be = _meta_call(router_logits)
    pos0 = pos0c.reshape(T)
    pos1 = pos1c.reshape(T)
    xs = _dispatch(x, pos0, pos1)
    ys = _mlp_call(be, xs, w1, w2)
    return _combine(ys, pos0, pos1, w0b, w1b)
